# baseline (speedup 1.0000x reference)
import jax
import jax.numpy as jnp
import numpy as np
from jax.experimental import pallas as pl

_TEMP = np.float32(0.7)


def _body(h_ref, w_ref, o_ref):
    h = h_ref[...]
    h_hi = h.astype(jnp.bfloat16)
    h_lo = (h - h_hi.astype(jnp.float32)).astype(jnp.bfloat16)
    w = w_ref[...].astype(jnp.bfloat16)
    acc = jnp.dot(h_hi, w, preferred_element_type=jnp.float32)
    acc += jnp.dot(h_lo, w, preferred_element_type=jnp.float32)
    o_ref[...] = acc / _TEMP


def kernel(hidden_states, gate_weight):
    B, S, D = hidden_states.shape
    E = gate_weight.shape[0]
    T = B * S
    h = hidden_states.reshape(T, D)
    wT = gate_weight.T

    TB = 1024
    out = pl.pallas_call(
        _body,
        grid=(T // TB,),
        in_specs=[
            pl.BlockSpec((TB, D), lambda i: (i, 0)),
            pl.BlockSpec((D, E), lambda i: (0, 0)),
        ],
        out_specs=pl.BlockSpec((TB, E), lambda i: (i, 0)),
        out_shape=jax.ShapeDtypeStruct((T, E), jnp.float32),
    )(h, wT)
    return out.reshape(B, S, E)


# speedup vs baseline: 1.1549x; 1.1549x over previous
import jax
import jax.numpy as jnp
import numpy as np
from jax.experimental import pallas as pl

_TEMP = np.float32(0.7)
_NSPLIT = 4


def _body(h0, h1, h2, h3, w_ref, o_ref):
    w = w_ref[...]
    D = w.shape[0]
    C = D // _NSPLIT
    acc = jnp.dot(h0[...], w[0 * C:1 * C], preferred_element_type=jnp.float32)
    acc += jnp.dot(h1[...], w[1 * C:2 * C], preferred_element_type=jnp.float32)
    acc += jnp.dot(h2[...], w[2 * C:3 * C], preferred_element_type=jnp.float32)
    acc += jnp.dot(h3[...], w[3 * C:4 * C], preferred_element_type=jnp.float32)
    o_ref[...] = acc / _TEMP


def kernel(hidden_states, gate_weight):
    B, S, D = hidden_states.shape
    E = gate_weight.shape[0]
    T = B * S
    h = hidden_states.reshape(T, D)
    wT = gate_weight.T

    TB = 1024
    C = D // _NSPLIT
    h_specs = [
        pl.BlockSpec((TB, C), lambda i, c=c: (i, c)) for c in range(_NSPLIT)
    ]
    out = pl.pallas_call(
        _body,
        grid=(T // TB,),
        in_specs=h_specs + [pl.BlockSpec((D, E), lambda i: (0, 0))],
        out_specs=pl.BlockSpec((TB, E), lambda i: (i, 0)),
        out_shape=jax.ShapeDtypeStruct((T, E), jnp.float32),
    )(*([h] * _NSPLIT), wT)
    return out.reshape(B, S, E)
